# baseline (speedup 1.0000x reference)
import jax
import jax.numpy as jnp
from jax import lax
from jax.experimental import pallas as pl

B = 8
NL = 13
L = 512
D = 768
NBINS = 320
DH = D // 2


def _body(vq_ref, xlo_ref, xhi_ref, o_ref):
    v = vq_ref[0]
    ixc = v[:, 0:1]
    iyc = v[:, 1:2]
    iota2 = lax.broadcasted_iota(jnp.int32, (L, 2 * NBINS), 1)
    is_x = iota2 < NBINS
    iota_mod = jnp.where(is_x, iota2, iota2 - NBINS)
    tgt = jnp.where(is_x, ixc, iyc)
    e = (tgt == iota_mod).astype(jnp.bfloat16)
    c = jnp.sum(e, axis=0, keepdims=True, dtype=jnp.float32)
    hi = c.astype(jnp.bfloat16)
    lo = (c - hi.astype(jnp.float32)).astype(jnp.bfloat16)
    hl = jnp.transpose(jnp.concatenate([hi, lo], axis=0))
    dn = (((1,), (0,)), ((), ()))
    fhl = lax.dot_general(e, hl, dn, preferred_element_type=jnp.float32)
    f = fhl[:, 0:1] + fhl[:, 1:2]
    r = 1.0 / f
    w = r * (1.0 / jnp.sum(r))
    olo = jnp.sum(xlo_ref[0, 0] * w, axis=0, keepdims=True)
    ohi = jnp.sum(xhi_ref[0, 0] * w, axis=0, keepdims=True)
    o_ref[0] = jnp.concatenate([olo, ohi], axis=1)


def kernel(input_feature, input_lengths, vq_indices):
    del input_lengths
    vq = vq_indices.astype(jnp.int32)
    out = pl.pallas_call(
        _body,
        grid=(B,),
        in_specs=[
            pl.BlockSpec((1, L, 2), lambda b: (b, 0, 0)),
            pl.BlockSpec((1, 1, L, DH), lambda b: (b, NL - 1, 0, 0)),
            pl.BlockSpec((1, 1, L, DH), lambda b: (b, NL - 1, 0, 1)),
        ],
        out_specs=pl.BlockSpec((1, 1, D), lambda b: (b, 0, 0)),
        out_shape=jax.ShapeDtypeStruct((B, 1, D), jnp.float32),
    )(vq, input_feature, input_feature)
    return out.reshape(B, D)


# speedup vs baseline: 1.1504x; 1.1504x over previous
import jax
import jax.numpy as jnp
from jax import lax
from jax.experimental import pallas as pl

B = 8
NL = 13
L = 512
D = 768
NBINS = 320
DH = D // 2


def _body(vq_ref, xlo_ref, xhi_ref, o_ref):
    v = vq_ref[0]
    ixr = v[0:1, :]
    iyr = v[1:2, :]
    iota_s = lax.broadcasted_iota(jnp.int32, (2 * NBINS, L), 0)
    is_x = iota_s < NBINS
    iota_mod = jnp.where(is_x, iota_s, iota_s - NBINS)
    tgt = jnp.where(is_x, ixr, iyr)
    et = (tgt == iota_mod).astype(jnp.bfloat16)
    ones_col = jnp.ones((L, 1), jnp.bfloat16)
    dn_nn = (((1,), (0,)), ((), ()))
    c = lax.dot_general(et, ones_col, dn_nn,
                        preferred_element_type=jnp.float32)
    hi = c.astype(jnp.bfloat16)
    lo = (c - hi.astype(jnp.float32)).astype(jnp.bfloat16)
    hl = jnp.concatenate([hi, lo], axis=1)
    dn_cc = (((0,), (0,)), ((), ()))
    fhl = lax.dot_general(hl, et, dn_cc,
                          preferred_element_type=jnp.float32)
    f = fhl[0:1, :] + fhl[1:2, :]
    r = 1.0 / f
    w = jnp.transpose(r * (1.0 / jnp.sum(r)))
    olo = jnp.sum(xlo_ref[0, 0] * w, axis=0, keepdims=True)
    ohi = jnp.sum(xhi_ref[0, 0] * w, axis=0, keepdims=True)
    o_ref[0] = jnp.concatenate([olo, ohi], axis=1)


def kernel(input_feature, input_lengths, vq_indices):
    del input_lengths
    vq = jnp.transpose(vq_indices.astype(jnp.int32), (0, 2, 1))
    out = pl.pallas_call(
        _body,
        grid=(B,),
        in_specs=[
            pl.BlockSpec((1, 2, L), lambda b: (b, 0, 0)),
            pl.BlockSpec((1, 1, L, DH), lambda b: (b, NL - 1, 0, 0)),
            pl.BlockSpec((1, 1, L, DH), lambda b: (b, NL - 1, 0, 1)),
        ],
        out_specs=pl.BlockSpec((1, 1, D), lambda b: (b, 0, 0)),
        out_shape=jax.ShapeDtypeStruct((B, 1, D), jnp.float32),
    )(vq, input_feature, input_feature)
    return out.reshape(B, D)
